# baseline (speedup 1.0000x reference)
import functools

import jax
import jax.numpy as jnp
from jax import lax
from jax.experimental import pallas as pl
from jax.experimental.pallas import tpu as pltpu
from jax.experimental.pallas import tpu_sc as plsc

EMBED_DIM = 64
CHUNK = 128
GROUP = 5
ROWS_G = GROUP * CHUNK


@functools.cache
def _build(n_total):
    info = plsc.get_sparse_core_info()
    nc, ns = info.num_cores, info.num_subcores
    nw = nc * ns
    rows_per_w = n_total // nw
    nchunk_w = rows_per_w // CHUNK
    ng = nchunk_w // GROUP
    assert rows_per_w * nw == n_total
    assert nchunk_w * CHUNK == rows_per_w
    assert ng * GROUP == nchunk_w and ng % 2 == 0 and ng >= 4

    mesh = plsc.VectorSubcoreMesh(core_axis_name="c", subcore_axis_name="s")

    @functools.partial(
        pl.kernel,
        mesh=mesh,
        out_type=jax.ShapeDtypeStruct((n_total, EMBED_DIM), jnp.float32),
        compiler_params=pltpu.CompilerParams(use_tc_tiling_on_sc=False),
        scratch_types=[
            pltpu.VMEM((nchunk_w, CHUNK), jnp.int32),
            pltpu.VMEM((ROWS_G, EMBED_DIM), jnp.float32),
            pltpu.VMEM((ROWS_G, EMBED_DIM), jnp.float32),
            pltpu.SemaphoreType.DMA,
            pltpu.SemaphoreType.DMA,
        ],
    )
    def gather_kernel(table_hbm, idx_hbm, out_hbm, idx_v, rows0, rows1, sg0, sg1):
        rows = [rows0, rows1]
        sg = [sg0, sg1]
        wid = lax.axis_index("s") * nc + lax.axis_index("c")
        out_base = wid * rows_per_w

        pltpu.sync_copy(idx_hbm.at[pl.ds(wid * nchunk_w, nchunk_w)], idx_v)

        def fire_g(g, b):
            for c in range(GROUP):
                pltpu.async_copy(
                    table_hbm.at[idx_v.at[g * GROUP + c]],
                    rows[b].at[pl.ds(c * CHUNK, CHUNK)],
                    sg[b],
                )

        def drain_g(b):
            for c in range(GROUP):
                pltpu.make_async_copy(
                    table_hbm.at[idx_v.at[c]],
                    rows[b].at[pl.ds(c * CHUNK, CHUNK)],
                    sg[b],
                ).wait()

        def write_out(g, b):
            pltpu.sync_copy(
                rows[b], out_hbm.at[pl.ds(out_base + g * ROWS_G, ROWS_G)]
            )

        fire_g(0, 0)
        fire_g(1, 1)

        def body(step, carry):
            for db in range(2):
                g = step * 2 + db
                b = db
                drain_g(b)
                write_out(g, b)
                fire_g(g + 2, b)
            return carry

        lax.fori_loop(0, (ng - 2) // 2, body, 0, unroll=False)

        drain_g(0)
        write_out(ng - 2, 0)
        drain_g(1)
        write_out(ng - 1, 1)

    return gather_kernel


def kernel(indices, table):
    batch, hist = indices.shape
    n_total = batch * hist
    idx2d = indices.astype(jnp.int32).reshape(n_total // CHUNK, CHUNK)
    out = _build(n_total)(table, idx2d)
    return out.reshape(batch, hist, table.shape[1])


# speedup vs baseline: 1.0004x; 1.0004x over previous
import functools

import jax
import jax.numpy as jnp
from jax import lax
from jax.experimental import pallas as pl
from jax.experimental.pallas import tpu as pltpu
from jax.experimental.pallas import tpu_sc as plsc

EMBED_DIM = 64
CHUNK = 128
NBUF = 8
PREF = 6


@functools.cache
def _build(n_total):
    info = plsc.get_sparse_core_info()
    nc, ns = info.num_cores, info.num_subcores
    nw = nc * ns
    rows_per_w = n_total // nw
    nchunk = rows_per_w // CHUNK
    assert rows_per_w * nw == n_total
    assert nchunk * CHUNK == rows_per_w
    assert nchunk > NBUF + PREF

    mesh = plsc.VectorSubcoreMesh(core_axis_name="c", subcore_axis_name="s")

    @functools.partial(
        pl.kernel,
        mesh=mesh,
        out_type=jax.ShapeDtypeStruct((n_total, EMBED_DIM), jnp.float32),
        compiler_params=pltpu.CompilerParams(use_tc_tiling_on_sc=False),
        scratch_types=[
            pltpu.VMEM((nchunk, CHUNK), jnp.int32),
            pltpu.VMEM((NBUF, CHUNK, EMBED_DIM), jnp.float32),
            [pltpu.SemaphoreType.DMA] * NBUF,
            [pltpu.SemaphoreType.DMA] * NBUF,
        ],
    )
    def gather_kernel(table_hbm, idx_hbm, out_hbm, idx_v, rows_v, sg, sw):
        wid = lax.axis_index("s") * nc + lax.axis_index("c")
        out_base = wid * rows_per_w

        pltpu.sync_copy(idx_hbm.at[pl.ds(wid * nchunk, nchunk)], idx_v)

        def fire_g(j, b):
            pltpu.async_copy(
                table_hbm.at[idx_v.at[j]], rows_v.at[b], sg[b]
            )

        def drain_g(b):
            pltpu.make_async_copy(
                table_hbm.at[idx_v.at[0]], rows_v.at[b], sg[b]
            ).wait()

        def fire_w(j, b):
            pltpu.async_copy(
                rows_v.at[b], out_hbm.at[pl.ds(out_base + j * CHUNK, CHUNK)], sw[b]
            )

        def drain_w(b):
            pltpu.make_async_copy(
                rows_v.at[b], out_hbm.at[pl.ds(out_base, CHUNK)], sw[b]
            ).wait()

        for j in range(PREF):
            fire_g(j, j)
        for j in range(NBUF):
            drain_g(j)
            fire_w(j, j)
            if j >= 2:
                drain_w((j + PREF) % NBUF)
            fire_g(j + PREF, (j + PREF) % NBUF)

        nrounds = nchunk // NBUF
        def body(r, carry):
            for db in range(NBUF):
                j = r * NBUF + db
                bn = (db + PREF) % NBUF
                drain_g(db)
                fire_w(j, db)
                drain_w(bn)
                fire_g(j + PREF, bn)
            return carry

        lax.fori_loop(1, nrounds - 1, body, 0, unroll=False)

        for db in range(NBUF):
            j = nchunk - NBUF + db
            drain_g(db)
            fire_w(j, db)
            if j + PREF < nchunk:
                drain_w((db + PREF) % NBUF)
                fire_g(j + PREF, (db + PREF) % NBUF)
        for b in range(NBUF):
            drain_w(b)

    return gather_kernel


def kernel(indices, table):
    batch, hist = indices.shape
    n_total = batch * hist
    idx2d = indices.astype(jnp.int32).reshape(n_total // CHUNK, CHUNK)
    out = _build(n_total)(table, idx2d)
    return out.reshape(batch, hist, table.shape[1])
